# Initial kernel scaffold; baseline (speedup 1.0000x reference)
import jax
import jax.numpy as jnp
from jax.experimental import pallas as pl


def kernel(x, edge_index, edge_attr, u, batch, params):
    raise NotImplementedError("write your pallas kernel here")


# speedup vs baseline: 6.4593x; 6.4593x over previous
import functools

import jax
import jax.numpy as jnp
from jax import lax
from jax.experimental import pallas as pl
from jax.experimental.pallas import tpu as pltpu
from jax.experimental.pallas import tpu_sc as plsc

F32 = jnp.float32

N = 50000
E = 800000
B = 256
DIM = 32

NC = 2
NS = 16
NW = NC * NS

NBLK = 2000
EBLK = 2000

EPW = E // NW
K2C = 200
K4C = 200
NPAD = 50048
ROWS_PT = NPAD // NS


def _relu(v):
    return jnp.maximum(v, 0.0)


def _dot(a, b):
    return jnp.dot(a, b, preferred_element_type=F32)


def _k1_body(x_ref, batch_ref, u_ref,
             wnd1_ref, bnd1_ref, wnd2_ref, bnd2_ref,
             wgd1_ref, bgd1_ref, wgd2_ref, bgd2_ref,
             w1a_ref, w1b_ref, w1d_ref, b1_ref,
             wn1b_ref, wn1c_ref, bn1_ref,
             tab_ref, nxu_ref, uh_ref):
    x = x_ref[...]
    xh = _dot(_relu(_dot(x, wnd1_ref[...]) + bnd1_ref[...]), wnd2_ref[...]) + bnd2_ref[...]
    uh = _dot(_relu(_dot(u_ref[...], wgd1_ref[...]) + bgd1_ref[...]), wgd2_ref[...]) + bgd2_ref[...]
    iota = lax.broadcasted_iota(jnp.int32, (x.shape[0], B), 1)
    oh = (batch_ref[...] == iota).astype(F32)
    t1 = _dot(uh, w1d_ref[...])
    t2 = _dot(uh, wn1c_ref[...])
    ga = _dot(xh, w1a_ref[...]) + _dot(oh, t1) + b1_ref[...]
    gb = _dot(xh, w1b_ref[...])
    tab_ref[0, :, :] = ga
    tab_ref[1, :, :] = gb
    nxu_ref[...] = _dot(xh, wn1b_ref[...]) + _dot(oh, t2) + bn1_ref[...]
    uh_ref[...] = uh


def _run_k1(x, batch2, u, wnd1, bnd1, wnd2, bnd2, wgd1, bgd1, wgd2, bgd2,
            w1a, w1b, w1d, b1, wn1b, wn1c, bn1):
    nblocks = N // NBLK
    full = lambda shape: pl.BlockSpec(shape, lambda i: (0,) * len(shape))
    return pl.pallas_call(
        _k1_body,
        grid=(nblocks,),
        in_specs=[
            pl.BlockSpec((NBLK, DIM), lambda i: (i, 0)),
            pl.BlockSpec((NBLK, 1), lambda i: (i, 0)),
            full((B, DIM)),
            full((DIM, 64)), full((1, 64)), full((64, DIM)), full((1, DIM)),
            full((DIM, 64)), full((1, 64)), full((64, DIM)), full((1, DIM)),
            full((DIM, 64)), full((DIM, 64)), full((DIM, 64)), full((1, 64)),
            full((DIM, 64)), full((DIM, 64)), full((1, 64)),
        ],
        out_specs=[
            pl.BlockSpec((2, NBLK, 64), lambda i: (0, i, 0)),
            pl.BlockSpec((NBLK, 64), lambda i: (i, 0)),
            pl.BlockSpec((B, DIM), lambda i: (0, 0)),
        ],
        out_shape=[
            jax.ShapeDtypeStruct((2, N, 64), F32),
            jax.ShapeDtypeStruct((N, 64), F32),
            jax.ShapeDtypeStruct((B, DIM), F32),
        ],
    )(x, batch2, u, wnd1, bnd1, wnd2, bnd2, wgd1, bgd1, wgd2, bgd2,
      w1a, w1b, w1d, b1, wn1b, wn1c, bn1)


def _k2_body(tab_hbm, row_hbm, colp_hbm, zeros16_hbm,
             g_hbm, cnt_hbm,
             idxr, idxc, bufa, bufb, ones_v, cnt_sh, sem_a, sem_b):
    cid = lax.axis_index("c")
    sid = lax.axis_index("s")
    wid = cid * NS + sid
    base = pl.multiple_of(wid * EPW, 8)

    def fill_ones(i, _):
        ones_v[i] = jnp.ones((16,), F32)
        return 0
    lax.fori_loop(0, K2C, fill_ones, 0)

    off = pl.multiple_of(sid * ROWS_PT, 8)
    pltpu.sync_copy(zeros16_hbm.at[pl.ds(off, ROWS_PT)], cnt_sh.at[pl.ds(off, ROWS_PT)])
    plsc.subcore_barrier()

    def chunk(i, _):
        b = pl.multiple_of(base + i * K2C, 8)
        pltpu.sync_copy(row_hbm.at[pl.ds(b, K2C)], idxr)
        pltpu.sync_copy(colp_hbm.at[pl.ds(b, K2C)], idxc)
        cpa = pltpu.async_copy(tab_hbm.at[idxr], bufa, sem_a)
        cpb = pltpu.async_copy(tab_hbm.at[idxc], bufb, sem_b)
        cpa.wait()
        cpb.wait()

        def addrow(r, _):
            for j in range(4):
                s = pl.ds(j * 16, 16)
                bufa[r, s] = bufa[r, s] + bufb[r, s]
            return 0
        lax.fori_loop(0, K2C, addrow, 0)

        pltpu.sync_copy(bufa, g_hbm.at[pl.ds(b, K2C)])
        pltpu.sync_copy(ones_v, cnt_sh.at[idxr], add=True)
        return 0
    lax.fori_loop(0, EPW // K2C, chunk, 0)

    plsc.subcore_barrier()
    pltpu.sync_copy(cnt_sh.at[pl.ds(off, ROWS_PT)], cnt_hbm.at[cid, pl.ds(off, ROWS_PT)])


def _run_k2(tab2, row, colp, zeros16):
    mesh = plsc.VectorSubcoreMesh(core_axis_name="c", subcore_axis_name="s")
    kfn = pl.kernel(
        _k2_body,
        out_type=[
            jax.ShapeDtypeStruct((E, 64), F32),
            jax.ShapeDtypeStruct((NC, NPAD, 16), F32),
        ],
        mesh=mesh,
        compiler_params=pltpu.CompilerParams(use_tc_tiling_on_sc=False),
        scratch_types=[
            pltpu.VMEM((K2C,), jnp.int32),
            pltpu.VMEM((K2C,), jnp.int32),
            pltpu.VMEM((K2C, 64), F32),
            pltpu.VMEM((K2C, 64), F32),
            pltpu.VMEM((K2C, 16), F32),
            pltpu.VMEM_SHARED((NPAD, 16), F32),
            pltpu.SemaphoreType.DMA,
            pltpu.SemaphoreType.DMA,
        ],
    )
    return kfn(tab2, row, colp, zeros16)


def _k3_body(ea_ref, g_ref,
             we1_ref, be1_ref, wc_ref, cc_ref, w2_ref, b2_ref, w3_ref, b3_ref,
             eo_ref):
    ea = ea_ref[...]
    pre1 = _relu(_dot(ea, we1_ref[...]) + be1_ref[...])
    h1 = _relu(g_ref[...] + _dot(pre1, wc_ref[...]) + cc_ref[...])
    h2 = _relu(_dot(h1, w2_ref[...]) + b2_ref[...])
    eo_ref[...] = _dot(h2, w3_ref[...]) + b3_ref[...] + ea


def _run_k3(edge_attr, g, we1, be1, wc, cc, w2, b2, w3, b3):
    nblocks = E // EBLK
    full = lambda shape: pl.BlockSpec(shape, lambda i: (0,) * len(shape))
    return pl.pallas_call(
        _k3_body,
        grid=(nblocks,),
        in_specs=[
            pl.BlockSpec((EBLK, DIM), lambda i: (i, 0)),
            pl.BlockSpec((EBLK, 64), lambda i: (i, 0)),
            full((DIM, 64)), full((1, 64)), full((64, 64)), full((1, 64)),
            full((64, 64)), full((1, 64)), full((64, DIM)), full((1, DIM)),
        ],
        out_specs=pl.BlockSpec((EBLK, DIM), lambda i: (i, 0)),
        out_shape=jax.ShapeDtypeStruct((E, DIM), F32),
    )(edge_attr, g, we1, be1, wc, cc, w2, b2, w3, b3)


def _k4_body(eo_hbm, row_hbm, zeros32_hbm,
             nsum_hbm,
             idx, data, acc_sh, sem):
    del sem
    cid = lax.axis_index("c")
    sid = lax.axis_index("s")
    wid = cid * NS + sid
    base = pl.multiple_of(wid * EPW, 8)
    off = pl.multiple_of(sid * ROWS_PT, 8)

    pltpu.sync_copy(zeros32_hbm.at[pl.ds(off, ROWS_PT)], acc_sh.at[pl.ds(off, ROWS_PT)])
    plsc.subcore_barrier()

    def chunk(i, _):
        b = pl.multiple_of(base + i * K4C, 8)
        pltpu.sync_copy(row_hbm.at[pl.ds(b, K4C)], idx)
        pltpu.sync_copy(eo_hbm.at[pl.ds(b, K4C)], data)
        pltpu.sync_copy(data, acc_sh.at[idx], add=True)
        return 0
    lax.fori_loop(0, EPW // K4C, chunk, 0)

    plsc.subcore_barrier()
    pltpu.sync_copy(acc_sh.at[pl.ds(off, ROWS_PT)], nsum_hbm.at[cid, pl.ds(off, ROWS_PT)])


def _run_k4(eo, row, zeros32):
    mesh = plsc.VectorSubcoreMesh(core_axis_name="c", subcore_axis_name="s")
    kfn = pl.kernel(
        _k4_body,
        out_type=jax.ShapeDtypeStruct((NC, NPAD, DIM), F32),
        mesh=mesh,
        compiler_params=pltpu.CompilerParams(use_tc_tiling_on_sc=False),
        scratch_types=[
            pltpu.VMEM((K4C,), jnp.int32),
            pltpu.VMEM((K4C, DIM), F32),
            pltpu.VMEM_SHARED((NPAD, DIM), F32),
            pltpu.SemaphoreType.DMA,
        ],
    )
    return kfn(eo, row, zeros32)


def _k5_body(sp_ref, cp_ref, nxu_ref, x_ref, batch_ref,
             wn1a_ref, wn2_ref, bn2_ref, wn3_ref, bn3_ref,
             xo_ref, bacc_ref):
    nsum = sp_ref[0, :, :] + sp_ref[1, :, :]
    cnt = cp_ref[0, :, 0:1] + cp_ref[1, :, 0:1]
    pooled = nsum / jnp.maximum(cnt, 1.0)
    h = _relu(_dot(pooled, wn1a_ref[...]) + nxu_ref[...])
    h2 = _relu(_dot(h, wn2_ref[...]) + bn2_ref[...])
    xo = _dot(h2, wn3_ref[...]) + bn3_ref[...] + x_ref[...]
    xo_ref[...] = xo

    nb = xo.shape[0]
    iota = lax.broadcasted_iota(jnp.int32, (nb, B), 1)
    oh = (batch_ref[...] == iota).astype(F32)
    ones = jnp.ones((nb, 1), F32)
    zeros = jnp.zeros((nb, 62), F32)
    payload = jnp.concatenate([xo, nsum, cnt, ones, zeros], axis=1)
    contrib = lax.dot_general(oh, payload, (((0,), (0,)), ((), ())),
                              preferred_element_type=F32)
    pid = pl.program_id(0)

    @pl.when(pid == 0)
    def _():
        bacc_ref[...] = contrib

    @pl.when(pid != 0)
    def _():
        bacc_ref[...] = bacc_ref[...] + contrib


def _run_k5(nsum_part, cnt_part, nxu, x, batch2, wn1a, wn2, bn2, wn3, bn3):
    nblocks = N // NBLK
    full = lambda shape: pl.BlockSpec(shape, lambda i: (0,) * len(shape))
    return pl.pallas_call(
        _k5_body,
        grid=(nblocks,),
        in_specs=[
            pl.BlockSpec((NC, NBLK, DIM), lambda i: (0, i, 0)),
            pl.BlockSpec((NC, NBLK, 16), lambda i: (0, i, 0)),
            pl.BlockSpec((NBLK, 64), lambda i: (i, 0)),
            pl.BlockSpec((NBLK, DIM), lambda i: (i, 0)),
            pl.BlockSpec((NBLK, 1), lambda i: (i, 0)),
            full((DIM, 64)), full((64, 64)), full((1, 64)),
            full((64, DIM)), full((1, DIM)),
        ],
        out_specs=[
            pl.BlockSpec((NBLK, DIM), lambda i: (i, 0)),
            pl.BlockSpec((B, 128), lambda i: (0, 0)),
        ],
        out_shape=[
            jax.ShapeDtypeStruct((N, DIM), F32),
            jax.ShapeDtypeStruct((B, 128), F32),
        ],
    )(nsum_part, cnt_part, nxu, x, batch2, wn1a, wn2, bn2, wn3, bn3)


def _k6_body(uh_ref, bacc_ref, u_ref,
             wg1a_ref, wg1b_ref, wg1c_ref, bg1_ref,
             wg2_ref, bg2_ref, wg3_ref, bg3_ref,
             uo_ref):
    bacc = bacc_ref[...]
    x_mean = bacc[:, 0:32] / jnp.maximum(bacc[:, 65:66], 1.0)
    edge_mean = bacc[:, 32:64] / jnp.maximum(bacc[:, 64:65], 1.0)
    h = _relu(_dot(uh_ref[...], wg1a_ref[...]) + _dot(x_mean, wg1b_ref[...])
              + _dot(edge_mean, wg1c_ref[...]) + bg1_ref[...])
    h2 = _relu(_dot(h, wg2_ref[...]) + bg2_ref[...])
    uo_ref[...] = _dot(h2, wg3_ref[...]) + bg3_ref[...] + u_ref[...]


def _run_k6(uh, bacc, u, wg1a, wg1b, wg1c, bg1, wg2, bg2, wg3, bg3):
    return pl.pallas_call(
        _k6_body,
        out_shape=jax.ShapeDtypeStruct((B, DIM), F32),
    )(uh, bacc, u, wg1a, wg1b, wg1c, bg1, wg2, bg2, wg3, bg3)


def kernel(x, edge_index, edge_attr, u, batch, params):
    (wnd1, bnd1), (wnd2, bnd2) = params["node_dense"]
    (wgd1, bgd1), (wgd2, bgd2) = params["global_dense"]
    (we1, be1), (we2, be2) = params["edge_dense"]
    (w1, b1), (w2, b2), (w3, b3) = params["edge_msg"]
    (wn1, bn1), (wn2, bn2), (wn3, bn3) = params["node_msg"]
    (wg1, bg1), (wg2, bg2), (wg3, bg3) = params["global_msg"]

    w1a, w1b, w1c, w1d = w1[0:32], w1[32:64], w1[64:96], w1[96:128]
    wn1a, wn1b, wn1c = wn1[0:32], wn1[32:64], wn1[64:96]
    wg1a, wg1b, wg1c = wg1[0:32], wg1[32:64], wg1[64:96]
    wc = we2 @ w1c
    cc = (be2 @ w1c)[None, :]

    r2 = lambda v: v[None, :]
    batch2 = batch[:, None]

    tab, nxu, uh = _run_k1(
        x, batch2, u,
        wnd1, r2(bnd1), wnd2, r2(bnd2),
        wgd1, r2(bgd1), wgd2, r2(bgd2),
        w1a, w1b, w1d, r2(b1), wn1b, wn1c, r2(bn1))
    tab2 = tab.reshape(2 * N, 64)

    row = edge_index[0]
    colp = edge_index[1] + N
    zeros16 = jnp.zeros((NPAD, 16), F32)
    zeros32 = jnp.zeros((NPAD, DIM), F32)

    g, cnt_part = _run_k2(tab2, row, colp, zeros16)
    eo = _run_k3(edge_attr, g, we1, r2(be1), wc, cc, w2, r2(b2), w3, r2(b3))
    nsum_part = _run_k4(eo, row, zeros32)
    xo, bacc = _run_k5(nsum_part, cnt_part, nxu, x, batch2,
                       wn1a, wn2, r2(bn2), wn3, r2(bn3))
    uo = _run_k6(uh, bacc, u, wg1a, wg1b, wg1c, r2(bg1), wg2, r2(bg2), wg3, r2(bg3))
    return xo, eo, uo


# speedup vs baseline: 8.4572x; 1.3093x over previous
import functools

import jax
import jax.numpy as jnp
from jax import lax
from jax.experimental import pallas as pl
from jax.experimental.pallas import tpu as pltpu
from jax.experimental.pallas import tpu_sc as plsc

F32 = jnp.float32

N = 50000
E = 800000
B = 256
DIM = 32

NC = 2
NS = 16
NW = NC * NS

NBLK = 2000
EBLK4 = 1000

EPW = E // NW
K2C = 200
K4C = 200
NPAD = 50048
ROWS_PT = NPAD // NS


def _relu(v):
    return jnp.maximum(v, 0.0)


def _dot(a, b):
    return jnp.dot(a, b, preferred_element_type=F32)


def _k1_body(x_ref, batch_ref, u_ref,
             wnd1_ref, bnd1_ref, wnd2_ref, bnd2_ref,
             wgd1_ref, bgd1_ref, wgd2_ref, bgd2_ref,
             w1a_ref, w1b_ref, w1d_ref, b1_ref,
             wn1b_ref, wn1c_ref, bn1_ref,
             tab_ref, nxu_ref, uh_ref):
    x = x_ref[...]
    xh = _dot(_relu(_dot(x, wnd1_ref[...]) + bnd1_ref[...]), wnd2_ref[...]) + bnd2_ref[...]
    uh = _dot(_relu(_dot(u_ref[...], wgd1_ref[...]) + bgd1_ref[...]), wgd2_ref[...]) + bgd2_ref[...]
    iota = lax.broadcasted_iota(jnp.int32, (x.shape[0], B), 1)
    oh = (batch_ref[...] == iota).astype(F32)
    t1 = _dot(uh, w1d_ref[...])
    t2 = _dot(uh, wn1c_ref[...])
    ga = _dot(xh, w1a_ref[...]) + _dot(oh, t1) + b1_ref[...]
    gb = _dot(xh, w1b_ref[...])
    tab_ref[0, :, :] = ga
    tab_ref[1, :, :] = gb
    nxu_ref[...] = _dot(xh, wn1b_ref[...]) + _dot(oh, t2) + bn1_ref[...]
    uh_ref[...] = uh


def _run_k1(x, batch2, u, wnd1, bnd1, wnd2, bnd2, wgd1, bgd1, wgd2, bgd2,
            w1a, w1b, w1d, b1, wn1b, wn1c, bn1):
    nblocks = N // NBLK
    full = lambda shape: pl.BlockSpec(shape, lambda i: (0,) * len(shape))
    return pl.pallas_call(
        _k1_body,
        grid=(nblocks,),
        in_specs=[
            pl.BlockSpec((NBLK, DIM), lambda i: (i, 0)),
            pl.BlockSpec((NBLK, 1), lambda i: (i, 0)),
            full((B, DIM)),
            full((DIM, 64)), full((1, 64)), full((64, DIM)), full((1, DIM)),
            full((DIM, 64)), full((1, 64)), full((64, DIM)), full((1, DIM)),
            full((DIM, 64)), full((DIM, 64)), full((DIM, 64)), full((1, 64)),
            full((DIM, 64)), full((DIM, 64)), full((1, 64)),
        ],
        out_specs=[
            pl.BlockSpec((2, NBLK, 64), lambda i: (0, i, 0)),
            pl.BlockSpec((NBLK, 64), lambda i: (i, 0)),
            pl.BlockSpec((B, DIM), lambda i: (0, 0)),
        ],
        out_shape=[
            jax.ShapeDtypeStruct((2, N, 64), F32),
            jax.ShapeDtypeStruct((N, 64), F32),
            jax.ShapeDtypeStruct((B, DIM), F32),
        ],
    )(x, batch2, u, wnd1, bnd1, wnd2, bnd2, wgd1, bgd1, wgd2, bgd2,
      w1a, w1b, w1d, b1, wn1b, wn1c, bn1)


def _k2_body(tab_hbm, row_hbm, colp_hbm, zeros16_hbm,
             g_hbm, cnt_hbm,
             idxr, idxc, bufa, bufb, out_v, ones_v, cnt_sh, sem_a, sem_b):
    cid = lax.axis_index("c")
    sid = lax.axis_index("s")
    wid = cid * NS + sid
    base = pl.multiple_of(wid * EPW, 8)

    def fill_ones(i, _):
        ones_v[i] = jnp.ones((16,), F32)
        return 0
    lax.fori_loop(0, K2C, fill_ones, 0)

    off = pl.multiple_of(sid * ROWS_PT, 8)
    pltpu.sync_copy(zeros16_hbm.at[pl.ds(off, ROWS_PT)], cnt_sh.at[pl.ds(off, ROWS_PT)])
    plsc.subcore_barrier()

    def chunk(i, _):
        b = pl.multiple_of(base + i * K2C, 8)
        pltpu.sync_copy(row_hbm.at[pl.ds(b, K2C)], idxr)
        pltpu.sync_copy(colp_hbm.at[pl.ds(b, K2C)], idxc)
        cpa = pltpu.async_copy(tab_hbm.at[idxr], bufa, sem_a)
        cpb = pltpu.async_copy(tab_hbm.at[idxc], bufb, sem_b)
        cpa.wait()
        cpb.wait()

        def addrow(r4, _):
            for sub in range(4):
                r = 4 * r4 + sub
                for j in range(4):
                    s = pl.ds(j * 16, 16)
                    out_v[r4, pl.ds(sub * 64 + j * 16, 16)] = bufa[r, s] + bufb[r, s]
            return 0
        lax.fori_loop(0, K2C // 4, addrow, 0)

        pltpu.sync_copy(out_v, g_hbm.at[pl.ds(b // 4, K2C // 4)])
        pltpu.sync_copy(ones_v, cnt_sh.at[idxr], add=True)
        return 0
    lax.fori_loop(0, EPW // K2C, chunk, 0)

    plsc.subcore_barrier()
    pltpu.sync_copy(cnt_sh.at[pl.ds(off, ROWS_PT)], cnt_hbm.at[cid, pl.ds(off, ROWS_PT)])


def _run_k2(tab2, row, colp, zeros16):
    mesh = plsc.VectorSubcoreMesh(core_axis_name="c", subcore_axis_name="s")
    kfn = pl.kernel(
        _k2_body,
        out_type=[
            jax.ShapeDtypeStruct((E // 4, 256), F32),
            jax.ShapeDtypeStruct((NC, NPAD, 16), F32),
        ],
        mesh=mesh,
        compiler_params=pltpu.CompilerParams(use_tc_tiling_on_sc=False),
        scratch_types=[
            pltpu.VMEM((K2C,), jnp.int32),
            pltpu.VMEM((K2C,), jnp.int32),
            pltpu.VMEM((K2C, 64), F32),
            pltpu.VMEM((K2C, 64), F32),
            pltpu.VMEM((K2C // 4, 256), F32),
            pltpu.VMEM((K2C, 16), F32),
            pltpu.VMEM_SHARED((NPAD, 16), F32),
            pltpu.SemaphoreType.DMA,
            pltpu.SemaphoreType.DMA,
        ],
    )
    return kfn(tab2, row, colp, zeros16)


def _k3_body(ea_ref, g_ref,
             we1_ref, be1_ref, wc_ref, cc_ref, w2_ref, b2_ref, w3_ref, b3_ref,
             eo_ref):
    ea = ea_ref[...]
    pre1 = _relu(_dot(ea, we1_ref[...]) + be1_ref[...])
    h1 = _relu(g_ref[...] + _dot(pre1, wc_ref[...]) + cc_ref[...])
    h2 = _relu(_dot(h1, w2_ref[...]) + b2_ref[...])
    eo_ref[...] = _dot(h2, w3_ref[...]) + b3_ref[...] + ea


def _run_k3(ea4, g4, we1d, be1d, wcd, ccd, w2d, b2d, w3d, b3d):
    nblocks = (E // 4) // EBLK4
    full = lambda shape: pl.BlockSpec(shape, lambda i: (0,) * len(shape))
    return pl.pallas_call(
        _k3_body,
        grid=(nblocks,),
        in_specs=[
            pl.BlockSpec((EBLK4, 128), lambda i: (i, 0)),
            pl.BlockSpec((EBLK4, 256), lambda i: (i, 0)),
            full((128, 256)), full((1, 256)), full((256, 256)), full((1, 256)),
            full((256, 256)), full((1, 256)), full((256, 128)), full((1, 128)),
        ],
        out_specs=pl.BlockSpec((EBLK4, 128), lambda i: (i, 0)),
        out_shape=jax.ShapeDtypeStruct((E // 4, 128), F32),
    )(ea4, g4, we1d, be1d, wcd, ccd, w2d, b2d, w3d, b3d)


def _k4_body(eo_hbm, row_hbm, zeros32_hbm,
             nsum_hbm,
             idx, data4, data, acc_sh, sem):
    del sem
    cid = lax.axis_index("c")
    sid = lax.axis_index("s")
    wid = cid * NS + sid
    base = pl.multiple_of(wid * EPW, 8)
    off = pl.multiple_of(sid * ROWS_PT, 8)

    pltpu.sync_copy(zeros32_hbm.at[pl.ds(off, ROWS_PT)], acc_sh.at[pl.ds(off, ROWS_PT)])
    plsc.subcore_barrier()

    def chunk(i, _):
        b = pl.multiple_of(base + i * K4C, 8)
        pltpu.sync_copy(row_hbm.at[pl.ds(b, K4C)], idx)
        pltpu.sync_copy(eo_hbm.at[pl.ds(b // 4, K4C // 4)], data4)
        def ung(r4, _):
            for sub in range(4):
                for j in range(2):
                    data[4 * r4 + sub, pl.ds(j * 16, 16)] = (
                        data4[r4, pl.ds(sub * 32 + j * 16, 16)])
            return 0
        lax.fori_loop(0, K4C // 4, ung, 0)
        pltpu.sync_copy(data, acc_sh.at[idx], add=True)
        return 0
    lax.fori_loop(0, EPW // K4C, chunk, 0)

    plsc.subcore_barrier()
    pltpu.sync_copy(acc_sh.at[pl.ds(off, ROWS_PT)], nsum_hbm.at[cid, pl.ds(off, ROWS_PT)])


def _run_k4(eo, row, zeros32):
    mesh = plsc.VectorSubcoreMesh(core_axis_name="c", subcore_axis_name="s")
    kfn = pl.kernel(
        _k4_body,
        out_type=jax.ShapeDtypeStruct((NC, NPAD, DIM), F32),
        mesh=mesh,
        compiler_params=pltpu.CompilerParams(use_tc_tiling_on_sc=False),
        scratch_types=[
            pltpu.VMEM((K4C,), jnp.int32),
            pltpu.VMEM((K4C // 4, 128), F32),
            pltpu.VMEM((K4C, DIM), F32),
            pltpu.VMEM_SHARED((NPAD, DIM), F32),
            pltpu.SemaphoreType.DMA,
        ],
    )
    return kfn(eo, row, zeros32)


def _k5_body(sp_ref, cp_ref, nxu_ref, x_ref, batch_ref,
             wn1a_ref, wn2_ref, bn2_ref, wn3_ref, bn3_ref,
             xo_ref, bacc_ref):
    nsum = sp_ref[0, :, :] + sp_ref[1, :, :]
    cnt = cp_ref[0, :, 0:1] + cp_ref[1, :, 0:1]
    pooled = nsum / jnp.maximum(cnt, 1.0)
    h = _relu(_dot(pooled, wn1a_ref[...]) + nxu_ref[...])
    h2 = _relu(_dot(h, wn2_ref[...]) + bn2_ref[...])
    xo = _dot(h2, wn3_ref[...]) + bn3_ref[...] + x_ref[...]
    xo_ref[...] = xo

    nb = xo.shape[0]
    iota = lax.broadcasted_iota(jnp.int32, (nb, B), 1)
    oh = (batch_ref[...] == iota).astype(F32)
    ones = jnp.ones((nb, 1), F32)
    zeros = jnp.zeros((nb, 62), F32)
    payload = jnp.concatenate([xo, nsum, cnt, ones, zeros], axis=1)
    contrib = lax.dot_general(oh, payload, (((0,), (0,)), ((), ())),
                              preferred_element_type=F32)
    pid = pl.program_id(0)

    @pl.when(pid == 0)
    def _():
        bacc_ref[...] = contrib

    @pl.when(pid != 0)
    def _():
        bacc_ref[...] = bacc_ref[...] + contrib


def _run_k5(nsum_part, cnt_part, nxu, x, batch2, wn1a, wn2, bn2, wn3, bn3):
    nblocks = N // NBLK
    full = lambda shape: pl.BlockSpec(shape, lambda i: (0,) * len(shape))
    return pl.pallas_call(
        _k5_body,
        grid=(nblocks,),
        in_specs=[
            pl.BlockSpec((NC, NBLK, DIM), lambda i: (0, i, 0)),
            pl.BlockSpec((NC, NBLK, 16), lambda i: (0, i, 0)),
            pl.BlockSpec((NBLK, 64), lambda i: (i, 0)),
            pl.BlockSpec((NBLK, DIM), lambda i: (i, 0)),
            pl.BlockSpec((NBLK, 1), lambda i: (i, 0)),
            full((DIM, 64)), full((64, 64)), full((1, 64)),
            full((64, DIM)), full((1, DIM)),
        ],
        out_specs=[
            pl.BlockSpec((NBLK, DIM), lambda i: (i, 0)),
            pl.BlockSpec((B, 128), lambda i: (0, 0)),
        ],
        out_shape=[
            jax.ShapeDtypeStruct((N, DIM), F32),
            jax.ShapeDtypeStruct((B, 128), F32),
        ],
    )(nsum_part, cnt_part, nxu, x, batch2, wn1a, wn2, bn2, wn3, bn3)


def _k6_body(uh_ref, bacc_ref, u_ref,
             wg1a_ref, wg1b_ref, wg1c_ref, bg1_ref,
             wg2_ref, bg2_ref, wg3_ref, bg3_ref,
             uo_ref):
    bacc = bacc_ref[...]
    x_mean = bacc[:, 0:32] / jnp.maximum(bacc[:, 65:66], 1.0)
    edge_mean = bacc[:, 32:64] / jnp.maximum(bacc[:, 64:65], 1.0)
    h = _relu(_dot(uh_ref[...], wg1a_ref[...]) + _dot(x_mean, wg1b_ref[...])
              + _dot(edge_mean, wg1c_ref[...]) + bg1_ref[...])
    h2 = _relu(_dot(h, wg2_ref[...]) + bg2_ref[...])
    uo_ref[...] = _dot(h2, wg3_ref[...]) + bg3_ref[...] + u_ref[...]


def _run_k6(uh, bacc, u, wg1a, wg1b, wg1c, bg1, wg2, bg2, wg3, bg3):
    return pl.pallas_call(
        _k6_body,
        out_shape=jax.ShapeDtypeStruct((B, DIM), F32),
    )(uh, bacc, u, wg1a, wg1b, wg1c, bg1, wg2, bg2, wg3, bg3)


def kernel(x, edge_index, edge_attr, u, batch, params):
    (wnd1, bnd1), (wnd2, bnd2) = params["node_dense"]
    (wgd1, bgd1), (wgd2, bgd2) = params["global_dense"]
    (we1, be1), (we2, be2) = params["edge_dense"]
    (w1, b1), (w2, b2), (w3, b3) = params["edge_msg"]
    (wn1, bn1), (wn2, bn2), (wn3, bn3) = params["node_msg"]
    (wg1, bg1), (wg2, bg2), (wg3, bg3) = params["global_msg"]

    w1a, w1b, w1c, w1d = w1[0:32], w1[32:64], w1[64:96], w1[96:128]
    wn1a, wn1b, wn1c = wn1[0:32], wn1[32:64], wn1[64:96]
    wg1a, wg1b, wg1c = wg1[0:32], wg1[32:64], wg1[64:96]
    wc = we2 @ w1c
    cc = be2 @ w1c

    eye4 = jnp.eye(4, dtype=F32)
    kd = lambda w: jnp.kron(eye4, w)
    t4 = lambda v: jnp.tile(v, 4)[None, :]

    r2 = lambda v: v[None, :]
    batch2 = batch[:, None]

    tab, nxu, uh = _run_k1(
        x, batch2, u,
        wnd1, r2(bnd1), wnd2, r2(bnd2),
        wgd1, r2(bgd1), wgd2, r2(bgd2),
        w1a, w1b, w1d, r2(b1), wn1b, wn1c, r2(bn1))
    tab2 = tab.reshape(2 * N, 64)

    row = edge_index[0]
    colp = edge_index[1] + N
    zeros16 = jnp.zeros((NPAD, 16), F32)
    zeros32 = jnp.zeros((NPAD, DIM), F32)

    ea4 = edge_attr.reshape(E // 4, 128)
    g4, cnt_part = _run_k2(tab2, row, colp, zeros16)
    eo4 = _run_k3(ea4, g4, kd(we1), t4(be1), kd(wc), t4(cc),
                  kd(w2), t4(b2), kd(w3), t4(b3))
    nsum_part = _run_k4(eo4, row, zeros32)
    xo, bacc = _run_k5(nsum_part, cnt_part, nxu, x, batch2,
                       wn1a, wn2, r2(bn2), wn3, r2(bn3))
    uo = _run_k6(uh, bacc, u, wg1a, wg1b, wg1c, r2(bg1), wg2, r2(bg2), wg3, r2(bg3))
    return xo, eo4.reshape(E, DIM), uo
